# baseline (speedup 1.0000x reference)
import functools

import jax
import jax.numpy as jnp
from jax import lax
from jax.experimental import pallas as pl
from jax.experimental.pallas import tpu as pltpu
from jax.experimental.pallas import tpu_sc as plsc

B, S, N, M, W = 16, 128, 4, 2048, 1
MC = 128
NCHUNK = M // MC // 2
TILE = 3 * MC * S
L = 16

_mesh = plsc.VectorSubcoreMesh(core_axis_name="c", subcore_axis_name="s")


@functools.partial(
    pl.kernel,
    mesh=_mesh,
    out_type=jax.ShapeDtypeStruct((B * 3 * M * S,), jnp.float32),
    scratch_types=[
        pltpu.VMEM((N, S), jnp.int32),
        pltpu.VMEM((TILE,), jnp.float32),
    ],
    compiler_params=pltpu.CompilerParams(needs_layout_passes=False),
)
def _sc_kernel(h_hbm, out_hbm, hv, buf):
    c = lax.axis_index("c")
    b = lax.axis_index("s")

    lanes = lax.iota(jnp.int32, L)
    vone = jnp.ones((L,), jnp.float32)
    vzero = jnp.zeros((L,), jnp.float32)

    def zero_body(t, _):
        for j in range(8):
            buf[pl.ds(t * 8 * L + j * L, L)] = vzero
        return 0

    lax.fori_loop(0, TILE // (8 * L), zero_body, 0)

    pltpu.sync_copy(h_hbm.at[b], hv)

    def scatter_phase(m0, add):
        val = vone if add else vzero
        op = plsc.addupdate_scatter if add else plsc.store_scatter
        for n in range(N):
            for j in range(S // L):
                svec = lanes + (j * L)
                h16 = hv[n, pl.ds(j * L, L)]
                rel = (h16 & (M - 1)) - m0
                inchunk = (rel >= 0) & (rel < MC)
                flat = rel * S + svec
                op(buf, [flat + 1], val, mask=inchunk & (svec < S - 1))
                op(buf, [flat + MC * S], val, mask=inchunk)
                op(buf, [flat + (2 * MC * S - 1)], val, mask=inchunk & (svec > 0))

    def chunk_body(i, _):
        m0 = (c * NCHUNK + i) * MC
        scatter_phase(m0, True)
        base = b * (3 * M * S) + m0 * S
        for k in range(3):
            pltpu.sync_copy(
                buf.at[pl.ds(k * MC * S, MC * S)],
                out_hbm.at[pl.ds(base + k * M * S, MC * S)],
            )
        scatter_phase(m0, False)
        return 0

    lax.fori_loop(0, NCHUNK, chunk_body, 0)


def kernel(sentencesMinHashes):
    h = jnp.transpose(sentencesMinHashes, (0, 2, 1))
    out = _sc_kernel(h)
    return out.reshape(B, 3 * M, S)


# speedup vs baseline: 1.0338x; 1.0338x over previous
import functools

import jax
import jax.numpy as jnp
from jax import lax
from jax.experimental import pallas as pl
from jax.experimental.pallas import tpu as pltpu
from jax.experimental.pallas import tpu_sc as plsc

B, S, N, M, W = 16, 128, 4, 2048, 1
MC = 128
NCHUNK = M // MC // 2
CS = MC * S
TILE = 3 * CS
L = 16

_mesh = plsc.VectorSubcoreMesh(core_axis_name="c", subcore_axis_name="s")


@functools.partial(
    pl.kernel,
    mesh=_mesh,
    out_type=jax.ShapeDtypeStruct((B * 3 * M * S,), jnp.float32),
    scratch_types=[
        pltpu.VMEM((N, S), jnp.int32),
        pltpu.VMEM((TILE,), jnp.float32),
        pltpu.VMEM((TILE,), jnp.float32),
        pltpu.SemaphoreType.DMA,
        pltpu.SemaphoreType.DMA,
    ],
    compiler_params=pltpu.CompilerParams(needs_layout_passes=False),
)
def _sc_kernel(h_hbm, out_hbm, hv, buf0, buf1, sem0, sem1):
    c = lax.axis_index("c")
    b = lax.axis_index("s")
    bufs = (buf0, buf1)
    sems = (sem0, sem1)

    lanes = lax.iota(jnp.int32, L)
    vone = jnp.ones((L,), jnp.float32)
    vzero = jnp.zeros((L,), jnp.float32)

    def zero_body(t, _):
        for buf in bufs:
            for j in range(8):
                buf[pl.ds(t * 8 * L + j * L, L)] = vzero
        return 0

    lax.fori_loop(0, TILE // (8 * L), zero_body, 0)

    pltpu.sync_copy(h_hbm.at[b], hv)

    def scatter_phase(buf, m0, add):
        val = vone if add else vzero
        op = plsc.addupdate_scatter if add else plsc.store_scatter
        for n in range(N):
            for j in range(S // L):
                svec = lanes + (j * L)
                h16 = hv[n, pl.ds(j * L, L)]
                rel = (h16 & (M - 1)) - m0
                inchunk = (rel >= 0) & (rel < MC)
                flat = rel * S + svec
                op(buf, [flat + 1], val, mask=inchunk & (svec < S - 1))
                op(buf, [flat + CS], val, mask=inchunk)
                op(buf, [flat + (2 * CS - 1)], val, mask=inchunk & (svec > 0))

    def m0_of(i):
        return (c * NCHUNK + i) * MC

    def dma(buf, sem, m0, fire):
        base = b * (3 * M * S) + m0 * S
        for k in range(3):
            cp = pltpu.make_async_copy(
                buf.at[pl.ds(k * CS, CS)],
                out_hbm.at[pl.ds(base + k * M * S, CS)],
                sem,
            )
            if fire:
                cp.start()
            else:
                cp.wait()

    def pair_body(j, _):
        for p in range(2):
            i = 2 * j + p
            m0 = m0_of(i)

            @pl.when(j > 0)
            def _():
                dma(bufs[p], sems[p], m0 - 2 * MC, fire=False)
                scatter_phase(bufs[p], m0 - 2 * MC, False)

            scatter_phase(bufs[p], m0, True)
            dma(bufs[p], sems[p], m0, fire=True)
        return 0

    lax.fori_loop(0, NCHUNK // 2, pair_body, 0)

    for p in range(2):
        dma(bufs[p], sems[p], m0_of(NCHUNK - 2 + p), fire=False)


def kernel(sentencesMinHashes):
    h = jnp.transpose(sentencesMinHashes, (0, 2, 1))
    out = _sc_kernel(h)
    return out.reshape(B, 3 * M, S)
